# baseline (speedup 1.0000x reference)
import functools

import jax
import jax.numpy as jnp
from jax import lax
from jax.experimental import pallas as pl
from jax.experimental.pallas import tpu as pltpu
from jax.experimental.pallas import tpu_sc as plsc

NUM_U = 20000
NUM_V = 30000
N = NUM_U + NUM_V
DIM = 64
HALF = 32
NC = 2
NS = 16
N_PAD = 50048
RPT = N_PAD // NS
TRASH = N
L = 64
KB = 14
GPT_POS = 784
GPT_NEG = 196
GP = NS * GPT_POS
GN = NS * GPT_NEG
ALPHA = 1.0 / 3.0

RPT_OUT = N // NS
RCH = 125
NCH = RPT_OUT // RCH


def _build_chain(body):
    mesh = plsc.VectorSubcoreMesh(
        core_axis_name="c", subcore_axis_name="s",
        num_cores=NC, num_subcores=NS)
    return functools.partial(
        pl.kernel,
        out_type=[jax.ShapeDtypeStruct((N, DIM), jnp.float32),
                  jax.ShapeDtypeStruct((2 * N_PAD, HALF), jnp.float32)],
        mesh=mesh,
        scratch_types=[
            pltpu.VMEM_SHARED((N_PAD, HALF), jnp.float32),
            pltpu.VMEM((KB, 2, L), jnp.int32),
            pltpu.VMEM((KB * L, HALF), jnp.float32),
            pltpu.SemaphoreType.DMA,
            pltpu.SemaphoreType.DMA,
        ],
        compiler_params=pltpu.CompilerParams(use_tc_tiling_on_sc=False),
    )(body)


def _chain_ops(acc, edgeb, rows, gsem, ssem):
    c = lax.axis_index("c")
    s = lax.axis_index("s")

    def init_from(tab):
        pltpu.sync_copy(tab.at[pl.ds(c * N_PAD + s * RPT, RPT)],
                        acc.at[pl.ds(s * RPT, RPT)])
        plsc.subcore_barrier()

    def writeback(out):
        pltpu.sync_copy(acc.at[pl.ds(s * RPT, RPT)],
                        out.at[pl.ds(c * N_PAD + s * RPT, RPT)])
        plsc.subcore_barrier()

    def seg(table, edges, gpt):
        gtot = edges.shape[0] // 2
        nb = gpt // KB

        def body(b, carry):
            g0 = s * gpt + b * KB
            pltpu.sync_copy(edges.at[pl.ds(c * gtot + g0, KB)], edgeb)
            gd = [pltpu.async_copy(
                table.at[edgeb.at[j, 0]], rows.at[pl.ds(j * L, L)], gsem)
                for j in range(KB)]
            sd = []
            for j in range(KB):
                gd[j].wait()
                sd.append(pltpu.async_copy(
                    rows.at[pl.ds(j * L, L)], acc.at[edgeb.at[j, 1]], ssem,
                    add=True))
            for d in sd:
                d.wait()
            return carry

        lax.fori_loop(0, nb, body, 0)
        plsc.subcore_barrier()

    def combine(base_tab, mid_tab, out):
        a, b2, d, o = 0, 128, 256, 384

        def chunk(t, carry):
            r0 = s * RPT_OUT + t * RCH
            pltpu.sync_copy(base_tab.at[pl.ds(c * N_PAD + r0, RCH)],
                            rows.at[pl.ds(a, RCH)])
            pltpu.sync_copy(mid_tab.at[pl.ds(c * N_PAD + r0, RCH)],
                            rows.at[pl.ds(b2, RCH)])
            pltpu.sync_copy(acc.at[pl.ds(r0, RCH)], rows.at[pl.ds(d, RCH)])

            def row(i, carry2):
                for h in (0, 16):
                    v = (rows[a + i, pl.ds(h, 16)]
                         + rows[b2 + i, pl.ds(h, 16)]
                         + rows[d + i, pl.ds(h, 16)]) * ALPHA
                    rows[o + i, pl.ds(h, 16)] = v
                return carry2

            lax.fori_loop(0, RCH, row, 0)
            pltpu.sync_copy(rows.at[pl.ds(o, RCH)],
                            out.at[pl.ds(r0, RCH), pl.ds(c * HALF, HALF)])
            return carry

        lax.fori_loop(0, NCH, chunk, 0)
        plsc.subcore_barrier()

    return init_from, writeback, seg, combine


def _pos_body(x0, sp, pos, xp1, acc, edgeb, rows, gsem, ssem):
    init_from, writeback, seg, combine = _chain_ops(acc, edgeb, rows,
                                                    gsem, ssem)
    init_from(x0)
    seg(x0, sp, GPT_POS)
    writeback(xp1)
    seg(xp1, sp, GPT_POS)
    combine(x0, xp1, pos)


def _neg_body(x0, en, sn, neg, xn1, acc, edgeb, rows, gsem, ssem):
    init_from, writeback, seg, combine = _chain_ops(acc, edgeb, rows,
                                                    gsem, ssem)
    init_from(x0)
    seg(x0, sn, GPT_NEG)
    writeback(xn1)
    seg(xn1, sn, GPT_NEG)
    combine(en, xn1, neg)


@functools.cache
def _build_pos():
    return _build_chain(_pos_body)


@functools.cache
def _build_neg():
    return _build_chain(_neg_body)


def _halves(x):
    z = jnp.zeros((N_PAD - N, HALF), jnp.float32)
    return jnp.concatenate([x[:, :HALF], z, x[:, HALF:], z], axis=0)


def _prep_edges(ei, gpt):
    tot = NS * gpt * L
    e = ei.shape[1]
    src = jnp.pad(ei[0], (0, tot - e)).reshape(tot // L, L)
    dst = jnp.pad(ei[1], (0, tot - e),
                  constant_values=TRASH).reshape(tot // L, L)
    both = jnp.stack([
        jnp.stack([src, dst], axis=1),
        jnp.stack([src + N_PAD, dst], axis=1),
    ])
    return both.reshape(2 * tot // L, 2, L)


def kernel(user_embedding, item_embedding, user_neg_embedding,
           item_neg_embedding, pos_edge_index, neg_edge_index):
    ego_pos = jnp.concatenate([user_embedding, item_embedding], axis=0)
    ego_neg = jnp.concatenate([user_neg_embedding, item_neg_embedding], axis=0)
    x0 = _halves(ego_pos)
    en = _halves(ego_neg)
    sp = _prep_edges(pos_edge_index, GPT_POS)
    sn = _prep_edges(neg_edge_index, GPT_NEG)

    pos, _ = _build_pos()(x0, sp)
    neg, _ = _build_neg()(x0, en, sn)
    return pos, neg


# speedup vs baseline: 1.0802x; 1.0802x over previous
import functools

import jax
import jax.numpy as jnp
from jax import lax
from jax.experimental import pallas as pl
from jax.experimental.pallas import tpu as pltpu
from jax.experimental.pallas import tpu_sc as plsc

NUM_U = 20000
NUM_V = 30000
N = NUM_U + NUM_V
DIM = 64
HALF = 32
NC = 2
NS = 16
N_PAD = 50048
RPT = N_PAD // NS
TRASH = N
L = 128
KB = 7
GPT_POS = 392
GPT_NEG = 98
GP = NS * GPT_POS
GN = NS * GPT_NEG
ALPHA = 1.0 / 3.0

RPT_OUT = N // NS
RCH = 125
NCH = RPT_OUT // RCH


def _build_chain(body):
    mesh = plsc.VectorSubcoreMesh(
        core_axis_name="c", subcore_axis_name="s",
        num_cores=NC, num_subcores=NS)
    return functools.partial(
        pl.kernel,
        out_type=[jax.ShapeDtypeStruct((N, DIM), jnp.float32),
                  jax.ShapeDtypeStruct((2 * N_PAD, HALF), jnp.float32)],
        mesh=mesh,
        scratch_types=[
            pltpu.VMEM_SHARED((N_PAD, HALF), jnp.float32),
            pltpu.VMEM((KB, 2, L), jnp.int32),
            pltpu.VMEM((KB * L, HALF), jnp.float32),
            pltpu.SemaphoreType.DMA,
            pltpu.SemaphoreType.DMA,
        ],
        compiler_params=pltpu.CompilerParams(use_tc_tiling_on_sc=False),
    )(body)


def _chain_ops(acc, edgeb, rows, gsem, ssem):
    c = lax.axis_index("c")
    s = lax.axis_index("s")

    def init_from(tab):
        pltpu.sync_copy(tab.at[pl.ds(c * N_PAD + s * RPT, RPT)],
                        acc.at[pl.ds(s * RPT, RPT)])
        plsc.subcore_barrier()

    def writeback(out):
        pltpu.sync_copy(acc.at[pl.ds(s * RPT, RPT)],
                        out.at[pl.ds(c * N_PAD + s * RPT, RPT)])
        plsc.subcore_barrier()

    def seg(table, edges, gpt):
        gtot = edges.shape[0] // 2
        nb = gpt // KB

        def body(b, carry):
            g0 = s * gpt + b * KB
            pltpu.sync_copy(edges.at[pl.ds(c * gtot + g0, KB)], edgeb)
            gd = [pltpu.async_copy(
                table.at[edgeb.at[j, 0]], rows.at[pl.ds(j * L, L)], gsem)
                for j in range(KB)]
            sd = []
            for j in range(KB):
                gd[j].wait()
                sd.append(pltpu.async_copy(
                    rows.at[pl.ds(j * L, L)], acc.at[edgeb.at[j, 1]], ssem,
                    add=True))
            for d in sd:
                d.wait()
            return carry

        lax.fori_loop(0, nb, body, 0)
        plsc.subcore_barrier()

    def combine(base_tab, mid_tab, out):
        a, b2, d, o = 0, 128, 256, 384

        def chunk(t, carry):
            r0 = s * RPT_OUT + t * RCH
            pltpu.sync_copy(base_tab.at[pl.ds(c * N_PAD + r0, RCH)],
                            rows.at[pl.ds(a, RCH)])
            pltpu.sync_copy(mid_tab.at[pl.ds(c * N_PAD + r0, RCH)],
                            rows.at[pl.ds(b2, RCH)])
            pltpu.sync_copy(acc.at[pl.ds(r0, RCH)], rows.at[pl.ds(d, RCH)])

            def row(i, carry2):
                for h in (0, 16):
                    v = (rows[a + i, pl.ds(h, 16)]
                         + rows[b2 + i, pl.ds(h, 16)]
                         + rows[d + i, pl.ds(h, 16)]) * ALPHA
                    rows[o + i, pl.ds(h, 16)] = v
                return carry2

            lax.fori_loop(0, RCH, row, 0)
            pltpu.sync_copy(rows.at[pl.ds(o, RCH)],
                            out.at[pl.ds(r0, RCH), pl.ds(c * HALF, HALF)])
            return carry

        lax.fori_loop(0, NCH, chunk, 0)
        plsc.subcore_barrier()

    return init_from, writeback, seg, combine


def _pos_body(x0, sp, pos, xp1, acc, edgeb, rows, gsem, ssem):
    init_from, writeback, seg, combine = _chain_ops(acc, edgeb, rows,
                                                    gsem, ssem)
    init_from(x0)
    seg(x0, sp, GPT_POS)
    writeback(xp1)
    seg(xp1, sp, GPT_POS)
    combine(x0, xp1, pos)


def _neg_body(x0, en, sn, neg, xn1, acc, edgeb, rows, gsem, ssem):
    init_from, writeback, seg, combine = _chain_ops(acc, edgeb, rows,
                                                    gsem, ssem)
    init_from(x0)
    seg(x0, sn, GPT_NEG)
    writeback(xn1)
    seg(xn1, sn, GPT_NEG)
    combine(en, xn1, neg)


@functools.cache
def _build_pos():
    return _build_chain(_pos_body)


@functools.cache
def _build_neg():
    return _build_chain(_neg_body)


def _halves(x):
    z = jnp.zeros((N_PAD - N, HALF), jnp.float32)
    return jnp.concatenate([x[:, :HALF], z, x[:, HALF:], z], axis=0)


def _prep_edges(ei, gpt):
    tot = NS * gpt * L
    e = ei.shape[1]
    src = jnp.pad(ei[0], (0, tot - e)).reshape(tot // L, L)
    dst = jnp.pad(ei[1], (0, tot - e),
                  constant_values=TRASH).reshape(tot // L, L)
    both = jnp.stack([
        jnp.stack([src, dst], axis=1),
        jnp.stack([src + N_PAD, dst], axis=1),
    ])
    return both.reshape(2 * tot // L, 2, L)


def kernel(user_embedding, item_embedding, user_neg_embedding,
           item_neg_embedding, pos_edge_index, neg_edge_index):
    ego_pos = jnp.concatenate([user_embedding, item_embedding], axis=0)
    ego_neg = jnp.concatenate([user_neg_embedding, item_neg_embedding], axis=0)
    x0 = _halves(ego_pos)
    en = _halves(ego_neg)
    sp = _prep_edges(pos_edge_index, GPT_POS)
    sn = _prep_edges(neg_edge_index, GPT_NEG)

    pos, _ = _build_pos()(x0, sp)
    neg, _ = _build_neg()(x0, en, sn)
    return pos, neg
